# Initial kernel scaffold; baseline (speedup 1.0000x reference)
import jax
import jax.numpy as jnp
from jax.experimental import pallas as pl


def kernel(x_nchw, w1, b1, w2, b2):
    raise NotImplementedError("write your pallas kernel here")


# speedup vs baseline: 1.1572x; 1.1572x over previous
import functools

import jax
import jax.numpy as jnp
from jax.experimental import pallas as pl
from jax.experimental.pallas import tpu as pltpu


def _se_fused_kernel(x_ref, w1_ref, b1_ref, w2_ref, b2_ref, o_ref, *, inv_hw):
    x = x_ref[...]
    pooled = jnp.sum(x, axis=1, keepdims=True, dtype=jnp.float32) * inv_hw
    h = jnp.dot(w1_ref[...], pooled,
                preferred_element_type=jnp.float32) + b1_ref[...]
    h = jnp.maximum(h, 0.0)
    s = jnp.dot(w2_ref[...], h,
                preferred_element_type=jnp.float32) + b2_ref[...]
    s = jax.nn.sigmoid(s)
    o_ref[...] = (x * s).astype(o_ref.dtype)


def kernel(x_nchw, w1, b1, w2, b2):
    N, C, H, W = x_nchw.shape
    R = w1.shape[0]
    HW = H * W
    x2 = x_nchw.reshape(N * C, HW)
    b1c = b1.reshape(R, 1).astype(jnp.float32)
    b2c = b2.reshape(C, 1).astype(jnp.float32)

    out = pl.pallas_call(
        functools.partial(_se_fused_kernel, inv_hw=1.0 / HW),
        out_shape=jax.ShapeDtypeStruct((N * C, HW), x_nchw.dtype),
        grid=(N,),
        in_specs=[
            pl.BlockSpec((C, HW), lambda n: (n, 0)),
            pl.BlockSpec((R, C), lambda n: (0, 0)),
            pl.BlockSpec((R, 1), lambda n: (0, 0)),
            pl.BlockSpec((C, R), lambda n: (0, 0)),
            pl.BlockSpec((C, 1), lambda n: (0, 0)),
        ],
        out_specs=pl.BlockSpec((C, HW), lambda n: (n, 0)),
        compiler_params=pltpu.CompilerParams(
            dimension_semantics=("parallel",),
            vmem_limit_bytes=64 * 1024 * 1024),
    )(x2, w1, b1c, w2, b2c)

    return out.reshape(N, C, H, W)


# speedup vs baseline: 8.1111x; 7.0091x over previous
import functools

import jax
import jax.numpy as jnp
from jax.experimental import pallas as pl
from jax.experimental.pallas import tpu as pltpu


def _se_fused_kernel(x_ref, w1t_ref, b1_ref, w2t_ref, b2_ref, o_ref, *, inv_hw):
    x = x_ref[...]
    pooled = jnp.sum(x, axis=0, keepdims=True,
                     dtype=jnp.float32) * inv_hw
    h = jnp.dot(pooled, w1t_ref[...],
                preferred_element_type=jnp.float32) + b1_ref[...]
    h = jnp.maximum(h, 0.0)
    s = jnp.dot(h, w2t_ref[...],
                preferred_element_type=jnp.float32) + b2_ref[...]
    s = jax.nn.sigmoid(s)
    o_ref[...] = (x * s).astype(o_ref.dtype)


def kernel(x_nchw, w1, b1, w2, b2):
    N, C, H, W = x_nchw.shape
    R = w1.shape[0]
    HW = H * W

    x2 = jnp.transpose(x_nchw, (0, 2, 3, 1)).reshape(N * HW, C)
    w1t = jnp.transpose(w1)
    w2t = jnp.transpose(w2)
    b1r = b1.reshape(1, R).astype(jnp.float32)
    b2r = b2.reshape(1, C).astype(jnp.float32)

    out = pl.pallas_call(
        functools.partial(_se_fused_kernel, inv_hw=1.0 / HW),
        out_shape=jax.ShapeDtypeStruct((N * HW, C), x_nchw.dtype),
        grid=(N,),
        in_specs=[
            pl.BlockSpec((HW, C), lambda n: (n, 0)),
            pl.BlockSpec((C, R), lambda n: (0, 0)),
            pl.BlockSpec((1, R), lambda n: (0, 0)),
            pl.BlockSpec((R, C), lambda n: (0, 0)),
            pl.BlockSpec((1, C), lambda n: (0, 0)),
        ],
        out_specs=pl.BlockSpec((HW, C), lambda n: (n, 0)),
        compiler_params=pltpu.CompilerParams(
            dimension_semantics=("parallel",),
            vmem_limit_bytes=64 * 1024 * 1024),
    )(x2, w1t, b1r, w2t, b2r)

    return jnp.transpose(out.reshape(N, H, W, C), (0, 3, 1, 2))


# speedup vs baseline: 8.2067x; 1.0118x over previous
import functools

import jax
import jax.numpy as jnp
from jax.experimental import pallas as pl
from jax.experimental.pallas import tpu as pltpu


def _se_fused_kernel(xa_ref, xb_ref, w1t_ref, b1_ref, w2t_ref, b2_ref,
                     o_ref, *, inv_hw):
    xa = xa_ref[...]
    xb = xb_ref[...]
    hh = xa.shape[0]
    pooled = (jnp.sum(xa, axis=0, keepdims=True, dtype=jnp.float32)
              + jnp.sum(xb, axis=0, keepdims=True, dtype=jnp.float32)) * inv_hw
    h = jnp.dot(pooled, w1t_ref[...],
                preferred_element_type=jnp.float32) + b1_ref[...]
    h = jnp.maximum(h, 0.0)
    s = jnp.dot(h, w2t_ref[...],
                preferred_element_type=jnp.float32) + b2_ref[...]
    s = jax.nn.sigmoid(s)
    o_ref[pl.ds(0, hh), :] = (xa * s).astype(o_ref.dtype)
    o_ref[pl.ds(hh, hh), :] = (xb * s).astype(o_ref.dtype)


def kernel(x_nchw, w1, b1, w2, b2):
    N, C, H, W = x_nchw.shape
    R = w1.shape[0]
    HW = H * W
    HH = HW // 2

    x2 = jnp.transpose(x_nchw, (0, 2, 3, 1)).reshape(N * HW, C)
    w1t = jnp.transpose(w1)
    w2t = jnp.transpose(w2)
    b1r = b1.reshape(1, R).astype(jnp.float32)
    b2r = b2.reshape(1, C).astype(jnp.float32)

    out = pl.pallas_call(
        functools.partial(_se_fused_kernel, inv_hw=1.0 / HW),
        out_shape=jax.ShapeDtypeStruct((N * HW, C), x_nchw.dtype),
        grid=(N,),
        in_specs=[
            pl.BlockSpec((HH, C), lambda n: (2 * n, 0)),
            pl.BlockSpec((HH, C), lambda n: (2 * n + 1, 0)),
            pl.BlockSpec((C, R), lambda n: (0, 0)),
            pl.BlockSpec((1, R), lambda n: (0, 0)),
            pl.BlockSpec((R, C), lambda n: (0, 0)),
            pl.BlockSpec((1, C), lambda n: (0, 0)),
        ],
        out_specs=pl.BlockSpec((HW, C), lambda n: (n, 0)),
        compiler_params=pltpu.CompilerParams(
            dimension_semantics=("parallel",),
            vmem_limit_bytes=64 * 1024 * 1024),
    )(x2, x2, w1t, b1r, w2t, b2r)

    return jnp.transpose(out.reshape(N, H, W, C), (0, 3, 1, 2))


# speedup vs baseline: 9.7605x; 1.1893x over previous
import functools

import jax
import jax.numpy as jnp
from jax.experimental import pallas as pl
from jax.experimental.pallas import tpu as pltpu


def _se_fused_kernel(x_ref, w1t_ref, b1_ref, w2t_ref, b2_ref, o_ref,
                     *, nb, hw, inv_hw):
    x = x_ref[...]
    c = x.shape[1]
    x3 = x.reshape(nb, hw, c)
    pooled = jnp.sum(x3, axis=1, dtype=jnp.float32) * inv_hw
    h = jnp.dot(pooled, w1t_ref[...],
                preferred_element_type=jnp.float32) + b1_ref[...]
    h = jnp.maximum(h, 0.0)
    s = jnp.dot(h, w2t_ref[...],
                preferred_element_type=jnp.float32) + b2_ref[...]
    s = jax.nn.sigmoid(s)
    o_ref[...] = (x3 * s[:, None, :]).reshape(nb * hw, c).astype(o_ref.dtype)


def kernel(x_nchw, w1, b1, w2, b2):
    N, C, H, W = x_nchw.shape
    R = w1.shape[0]
    HW = H * W
    NB = 2 if N % 2 == 0 else 1

    x2 = jnp.transpose(x_nchw, (0, 2, 3, 1)).reshape(N * HW, C)
    w1t = jnp.transpose(w1)
    w2t = jnp.transpose(w2)
    b1r = b1.reshape(1, R).astype(jnp.float32)
    b2r = b2.reshape(1, C).astype(jnp.float32)

    out = pl.pallas_call(
        functools.partial(_se_fused_kernel, nb=NB, hw=HW, inv_hw=1.0 / HW),
        out_shape=jax.ShapeDtypeStruct((N * HW, C), x_nchw.dtype),
        grid=(N // NB,),
        in_specs=[
            pl.BlockSpec((NB * HW, C), lambda n: (n, 0)),
            pl.BlockSpec((C, R), lambda n: (0, 0)),
            pl.BlockSpec((1, R), lambda n: (0, 0)),
            pl.BlockSpec((R, C), lambda n: (0, 0)),
            pl.BlockSpec((1, C), lambda n: (0, 0)),
        ],
        out_specs=pl.BlockSpec((NB * HW, C), lambda n: (n, 0)),
        compiler_params=pltpu.CompilerParams(
            dimension_semantics=("parallel",),
            vmem_limit_bytes=64 * 1024 * 1024),
    )(x2, w1t, b1r, w2t, b2r)

    return jnp.transpose(out.reshape(N, H, W, C), (0, 3, 1, 2))


# speedup vs baseline: 10.1910x; 1.0441x over previous
import functools

import jax
import jax.numpy as jnp
from jax.experimental import pallas as pl
from jax.experimental.pallas import tpu as pltpu


def _se_fused_kernel(x_ref, w1t_ref, b1_ref, w2t_ref, b2_ref, o_ref,
                     *, nb, hw, inv_hw):
    x = x_ref[...]
    c = x.shape[1]
    x3 = x.reshape(nb, hw, c)
    pooled = jnp.sum(x3, axis=1, dtype=jnp.float32) * inv_hw
    h = jnp.dot(pooled, w1t_ref[...],
                preferred_element_type=jnp.float32) + b1_ref[...]
    h = jnp.maximum(h, 0.0)
    s = jnp.dot(h, w2t_ref[...],
                preferred_element_type=jnp.float32) + b2_ref[...]
    s = jax.nn.sigmoid(s)
    o_ref[...] = (x3 * s[:, None, :]).reshape(nb * hw, c).astype(o_ref.dtype)


def kernel(x_nchw, w1, b1, w2, b2):
    N, C, H, W = x_nchw.shape
    R = w1.shape[0]
    HW = H * W
    NB = 4 if N % 4 == 0 else (2 if N % 2 == 0 else 1)

    x2 = jnp.transpose(x_nchw, (0, 2, 3, 1)).reshape(N * HW, C)
    w1t = jnp.transpose(w1)
    w2t = jnp.transpose(w2)
    b1r = b1.reshape(1, R).astype(jnp.float32)
    b2r = b2.reshape(1, C).astype(jnp.float32)

    out = pl.pallas_call(
        functools.partial(_se_fused_kernel, nb=NB, hw=HW, inv_hw=1.0 / HW),
        out_shape=jax.ShapeDtypeStruct((N * HW, C), x_nchw.dtype),
        grid=(N // NB,),
        in_specs=[
            pl.BlockSpec((NB * HW, C), lambda n: (n, 0)),
            pl.BlockSpec((C, R), lambda n: (0, 0)),
            pl.BlockSpec((1, R), lambda n: (0, 0)),
            pl.BlockSpec((R, C), lambda n: (0, 0)),
            pl.BlockSpec((1, C), lambda n: (0, 0)),
        ],
        out_specs=pl.BlockSpec((NB * HW, C), lambda n: (n, 0)),
        compiler_params=pltpu.CompilerParams(
            dimension_semantics=("parallel",),
            vmem_limit_bytes=64 * 1024 * 1024),
    )(x2, w1t, b1r, w2t, b2r)

    return jnp.transpose(out.reshape(N, H, W, C), (0, 3, 1, 2))


# speedup vs baseline: 10.1965x; 1.0005x over previous
import functools

import jax
import jax.numpy as jnp
from jax.experimental import pallas as pl
from jax.experimental.pallas import tpu as pltpu


def _se_fused_kernel(xa_ref, xb_ref, w1t_ref, b1_ref, w2t_ref, b2_ref, o_ref,
                     *, nb, hw, inv_hw):
    xa = xa_ref[...]
    xb = xb_ref[...]
    c = xa.shape[1]
    half = nb // 2
    x3a = xa.reshape(half, hw, c)
    x3b = xb.reshape(half, hw, c)
    pooled = jnp.concatenate(
        [jnp.sum(x3a, axis=1, dtype=jnp.float32),
         jnp.sum(x3b, axis=1, dtype=jnp.float32)], axis=0) * inv_hw
    h = jnp.dot(pooled, w1t_ref[...],
                preferred_element_type=jnp.float32) + b1_ref[...]
    h = jnp.maximum(h, 0.0)
    s = jnp.dot(h, w2t_ref[...],
                preferred_element_type=jnp.float32) + b2_ref[...]
    s = jax.nn.sigmoid(s)
    rows = half * hw
    o_ref[pl.ds(0, rows), :] = (
        x3a * s[:half, None, :]).reshape(rows, c).astype(o_ref.dtype)
    o_ref[pl.ds(rows, rows), :] = (
        x3b * s[half:, None, :]).reshape(rows, c).astype(o_ref.dtype)


def _se_fused_kernel_nb1(x_ref, w1t_ref, b1_ref, w2t_ref, b2_ref, o_ref,
                         *, inv_hw):
    x = x_ref[...]
    pooled = jnp.sum(x, axis=0, keepdims=True,
                     dtype=jnp.float32) * inv_hw
    h = jnp.dot(pooled, w1t_ref[...],
                preferred_element_type=jnp.float32) + b1_ref[...]
    h = jnp.maximum(h, 0.0)
    s = jnp.dot(h, w2t_ref[...],
                preferred_element_type=jnp.float32) + b2_ref[...]
    s = jax.nn.sigmoid(s)
    o_ref[...] = (x * s).astype(o_ref.dtype)


def kernel(x_nchw, w1, b1, w2, b2):
    N, C, H, W = x_nchw.shape
    R = w1.shape[0]
    HW = H * W
    NB = 4 if N % 4 == 0 else (2 if N % 2 == 0 else 1)

    x2 = jnp.transpose(x_nchw, (0, 2, 3, 1)).reshape(N * HW, C)
    w1t = jnp.transpose(w1)
    w2t = jnp.transpose(w2)
    b1r = b1.reshape(1, R).astype(jnp.float32)
    b2r = b2.reshape(1, C).astype(jnp.float32)

    wspecs = [
        pl.BlockSpec((C, R), lambda n: (0, 0)),
        pl.BlockSpec((1, R), lambda n: (0, 0)),
        pl.BlockSpec((R, C), lambda n: (0, 0)),
        pl.BlockSpec((1, C), lambda n: (0, 0)),
    ]
    common = dict(
        out_shape=jax.ShapeDtypeStruct((N * HW, C), x_nchw.dtype),
        grid=(N // NB,),
        out_specs=pl.BlockSpec((NB * HW, C), lambda n: (n, 0)),
        compiler_params=pltpu.CompilerParams(
            dimension_semantics=("parallel",),
            vmem_limit_bytes=64 * 1024 * 1024),
    )

    if NB == 1:
        out = pl.pallas_call(
            functools.partial(_se_fused_kernel_nb1, inv_hw=1.0 / HW),
            in_specs=[pl.BlockSpec((HW, C), lambda n: (n, 0))] + wspecs,
            **common,
        )(x2, w1t, b1r, w2t, b2r)
    else:
        hrows = (NB // 2) * HW
        out = pl.pallas_call(
            functools.partial(_se_fused_kernel, nb=NB, hw=HW, inv_hw=1.0 / HW),
            in_specs=[
                pl.BlockSpec((hrows, C), lambda n: (2 * n, 0)),
                pl.BlockSpec((hrows, C), lambda n: (2 * n + 1, 0)),
            ] + wspecs,
            **common,
        )(x2, x2, w1t, b1r, w2t, b2r)

    return jnp.transpose(out.reshape(N, H, W, C), (0, 3, 1, 2))
